# Initial kernel scaffold; baseline (speedup 1.0000x reference)
import jax
import jax.numpy as jnp
from jax.experimental import pallas as pl


def kernel(x, adj, W1, b1, W2, b2, W3, b3):
    raise NotImplementedError("write your pallas kernel here")


# speedup vs baseline: 1.0203x; 1.0203x over previous
import functools

import jax
import jax.numpy as jnp
from jax.experimental import pallas as pl
from jax.experimental.pallas import tpu as pltpu


def _gcn_kernel(x_ref, adj_ref, w1_ref, w2_ref, w3_ref, b1_ref, b2_ref,
                b3_ref, out_ref, sup_a, sup_b, *, bm):
    l = pl.program_id(0)
    i = pl.program_id(1)

    @pl.when((l == 0) & (i == 0))
    def _init():
        sup_a[...] = jnp.dot(
            x_ref[...], w1_ref[...],
            preferred_element_type=jnp.float32).astype(jnp.bfloat16)

    a_strip = adj_ref[...].astype(jnp.bfloat16)

    @pl.when(l == 0)
    def _layer0():
        h = jnp.dot(a_strip, sup_a[...],
                    preferred_element_type=jnp.float32) + b1_ref[...]
        h = jnp.maximum(h, 0.0).astype(jnp.bfloat16)
        sup_b[pl.ds(i * bm, bm), :] = jnp.dot(
            h, w2_ref[...], preferred_element_type=jnp.float32
        ).astype(jnp.bfloat16)

    @pl.when(l == 1)
    def _layer1():
        h = jnp.dot(a_strip, sup_b[...],
                    preferred_element_type=jnp.float32) + b2_ref[...]
        h = jnp.maximum(h, 0.0).astype(jnp.bfloat16)
        sup_a[pl.ds(i * bm, bm), :] = jnp.dot(
            h, w3_ref[...], preferred_element_type=jnp.float32
        ).astype(jnp.bfloat16)

    @pl.when(l == 2)
    def _layer2():
        out_ref[...] = jnp.dot(
            a_strip, sup_a[...],
            preferred_element_type=jnp.float32) + b3_ref[...]


def kernel(x, adj, W1, b1, W2, b2, W3, b3):
    n, d_in = x.shape
    d_out = W3.shape[1]
    bm = 400 if n % 400 == 0 else n
    nb = n // bm

    xb = x.astype(jnp.bfloat16)
    w1b = W1.astype(jnp.bfloat16)
    w2b = W2.astype(jnp.bfloat16)
    w3b = W3.astype(jnp.bfloat16)
    b1r = b1.reshape(1, -1)
    b2r = b2.reshape(1, -1)
    b3r = b3.reshape(1, -1)

    full = lambda shape: pl.BlockSpec(shape, lambda l, i: (0, 0))
    return pl.pallas_call(
        functools.partial(_gcn_kernel, bm=bm),
        grid=(3, nb),
        in_specs=[
            full((n, d_in)),
            pl.BlockSpec((bm, n), lambda l, i: (i, 0)),
            full(W1.shape), full(W2.shape), full(W3.shape),
            full((1, d_in)), full((1, d_in)), full((1, d_out)),
        ],
        out_specs=pl.BlockSpec((bm, d_out), lambda l, i: (i, 0)),
        out_shape=jax.ShapeDtypeStruct((n, d_out), jnp.float32),
        scratch_shapes=[
            pltpu.VMEM((n, W1.shape[1]), jnp.bfloat16),
            pltpu.VMEM((n, W2.shape[1]), jnp.bfloat16),
        ],
        compiler_params=pltpu.CompilerParams(
            dimension_semantics=("arbitrary", "arbitrary")),
    )(xb, adj, w1b, w2b, w3b, b1r, b2r, b3r)
